# baseline (speedup 1.0000x reference)
import functools

import jax
import jax.numpy as jnp
from jax import lax
from jax.experimental import pallas as pl
from jax.experimental.pallas import tpu as pltpu
from jax.experimental.pallas import tpu_sc as plsc

_ROWS, _COLS = 16384, 200
_NC, _NS = 2, 16
_NW = _NC * _NS
_RPW = _ROWS // _NW
_CH = 32
_NCHUNK = _RPW // _CH
_PCOLS = 208


def _sc_body(ts_hbm, lab_hbm, out_hbm, t_buf, l_buf, o_buf, sem_t, sem_l, sem_o):
    wid = lax.axis_index("s") * _NC + lax.axis_index("c")
    base_row = wid * _RPW
    iota = lax.iota(jnp.int32, 16)
    zero = jnp.zeros((16,), jnp.int32)
    one = jnp.ones((16,), jnp.int32)

    def chunk(ci, carry):
        r0 = base_row + ci * _CH
        ct = pltpu.make_async_copy(
            ts_hbm.at[pl.ds(r0, _CH), :], t_buf.at[:, pl.ds(0, _COLS)], sem_t)
        cl = pltpu.make_async_copy(
            lab_hbm.at[pl.ds(r0, _CH), :], l_buf.at[:, pl.ds(0, _COLS)], sem_l)
        ct.start()
        cl.start()
        ct.wait()
        cl.wait()

        def row(r, rcarry):
            idx_r = jnp.full((16,), r, jnp.int32)
            for w in range(13):
                off = 16 * w
                idx_c = iota + off
                mask = None if w < 12 else (idx_c < _COLS)
                vt = t_buf[r, pl.ds(off, 16)]
                plsc.store_scatter(o_buf, [idx_r, idx_c, zero], vt, mask=mask)
                vl = l_buf[r, pl.ds(off, 16)]
                plsc.store_scatter(o_buf, [idx_r, idx_c, one], vl, mask=mask)
            return rcarry

        lax.fori_loop(0, _CH, row, 0)
        co = pltpu.make_async_copy(o_buf, out_hbm.at[pl.ds(r0, _CH)], sem_o)
        co.start()
        co.wait()
        return carry

    lax.fori_loop(0, _NCHUNK, chunk, 0)


def kernel(timestamps, labels, seq_lens):
    mesh = plsc.VectorSubcoreMesh(
        core_axis_name="c", subcore_axis_name="s",
        num_cores=_NC, num_subcores=_NS)
    payload = pl.kernel(
        _sc_body,
        out_type=jax.ShapeDtypeStruct((_ROWS, _COLS, 2), timestamps.dtype),
        mesh=mesh,
        compiler_params=pltpu.CompilerParams(
            use_tc_tiling_on_sc=False, needs_layout_passes=False),
        scratch_types=[
            pltpu.VMEM((_CH, _PCOLS), jnp.float32),
            pltpu.VMEM((_CH, _PCOLS), jnp.float32),
            pltpu.VMEM((_CH, _COLS, 2), jnp.float32),
            pltpu.SemaphoreType.DMA,
            pltpu.SemaphoreType.DMA,
            pltpu.SemaphoreType.DMA,
        ],
    )(timestamps, labels)
    return (payload, seq_lens)


# speedup vs baseline: 9.2095x; 9.2095x over previous
import functools

import jax
import jax.numpy as jnp
from jax import lax
from jax.experimental import pallas as pl
from jax.experimental.pallas import tpu as pltpu
from jax.experimental.pallas import tpu_sc as plsc

_ROWS, _COLS = 16384, 200
_OCOLS = 2 * _COLS
_NC, _NS = 2, 16
_NW = _NC * _NS
_RPW = _ROWS // _NW
_CH = 32
_NCHUNK = _RPW // _CH


def _sc_body(ts_hbm, lab_hbm, out_hbm, t_buf, l_buf, o_buf, sem_t, sem_l, sem_o):
    wid = lax.axis_index("s") * _NC + lax.axis_index("c")
    base_row = wid * _RPW
    iota = lax.iota(jnp.int32, 16)

    def chunk(ci, carry):
        r0 = base_row + ci * _CH
        ct = pltpu.make_async_copy(
            ts_hbm.at[pl.ds(r0, _CH), :], t_buf, sem_t)
        cl = pltpu.make_async_copy(
            lab_hbm.at[pl.ds(r0, _CH), :], l_buf, sem_l)
        ct.start()
        cl.start()
        ct.wait()
        cl.wait()

        def row(r, rcarry):
            idx_r = jnp.full((16,), r, jnp.int32)
            for off in [16 * w for w in range(12)] + [_COLS - 16]:
                src_c = iota + off
                idx_t = 2 * src_c
                idx_l = 2 * src_c + 1
                vt = t_buf[r, pl.ds(off, 16)]
                plsc.store_scatter(o_buf, [idx_r, idx_t], vt)
                vl = l_buf[r, pl.ds(off, 16)]
                plsc.store_scatter(o_buf, [idx_r, idx_l], vl)
            return rcarry

        lax.fori_loop(0, _CH, row, 0)
        co = pltpu.make_async_copy(o_buf, out_hbm.at[pl.ds(r0, _CH), :], sem_o)
        co.start()
        co.wait()
        return carry

    lax.fori_loop(0, _NCHUNK, chunk, 0)


def kernel(timestamps, labels, seq_lens):
    mesh = plsc.VectorSubcoreMesh(
        core_axis_name="c", subcore_axis_name="s",
        num_cores=_NC, num_subcores=_NS)
    flat = pl.kernel(
        _sc_body,
        out_type=jax.ShapeDtypeStruct((_ROWS, _OCOLS), timestamps.dtype),
        mesh=mesh,
        compiler_params=pltpu.CompilerParams(
            use_tc_tiling_on_sc=True, needs_layout_passes=False),
        scratch_types=[
            pltpu.VMEM((_CH, _COLS), jnp.float32),
            pltpu.VMEM((_CH, _COLS), jnp.float32),
            pltpu.VMEM((_CH, _OCOLS), jnp.float32),
            pltpu.SemaphoreType.DMA,
            pltpu.SemaphoreType.DMA,
            pltpu.SemaphoreType.DMA,
        ],
    )(timestamps, labels)
    payload = flat.reshape(_ROWS, _COLS, 2)
    return (payload, seq_lens)
